# baseline (speedup 1.0000x reference)
import functools

import jax
import jax.numpy as jnp
from jax import lax
from jax.experimental import pallas as pl
from jax.experimental.pallas import tpu as pltpu
from jax.experimental.pallas import tpu_sc as plsc

VOCAB = 100000
DIM = 128
RANK = 256
SCALING = 512.0 / 256.0
N_TOK = 1024 * 20
HALF = N_TOK // 2

NC = 2
NS = 16
NW = NC * NS
TOK_PER_W = HALF // NW
CKT = 80
NCKT = TOK_PER_W // CKT

_mesh = plsc.VectorSubcoreMesh(core_axis_name="c", subcore_axis_name="s")


def _pipelined_gather(table_hbm, idx_v, out_hbm, base, bufs, gsems, ssems):
    g = {}
    s = {}
    g[0] = pltpu.async_copy(
        table_hbm.at[idx_v.at[pl.ds(0, CKT)]], bufs.at[0], gsems[0]
    )
    for c in range(NCKT):
        p = c % 2
        if c + 1 < NCKT:
            if c >= 1:
                s[c - 1].wait()
            g[c + 1] = pltpu.async_copy(
                table_hbm.at[idx_v.at[pl.ds((c + 1) * CKT, CKT)]],
                bufs.at[1 - p],
                gsems[1 - p],
            )
        g[c].wait()
        s[c] = pltpu.async_copy(
            bufs.at[p], out_hbm.at[pl.ds(base + c * CKT, CKT)], ssems[p]
        )
    s[NCKT - 2].wait()
    s[NCKT - 1].wait()


@functools.partial(
    pl.kernel,
    mesh=_mesh,
    out_type=(
        jax.ShapeDtypeStruct((HALF, RANK), jnp.float32),
        jax.ShapeDtypeStruct((HALF, DIM), jnp.float32),
    ),
    compiler_params=pltpu.CompilerParams(use_tc_tiling_on_sc=True),
    scratch_types=[
        pltpu.VMEM((TOK_PER_W,), jnp.int32),
        pltpu.VMEM((2, CKT, RANK), jnp.float32),
        pltpu.VMEM((2, CKT, DIM), jnp.float32),
        pltpu.SemaphoreType.DMA,
        pltpu.SemaphoreType.DMA,
        pltpu.SemaphoreType.DMA,
        pltpu.SemaphoreType.DMA,
    ],
)
def _sc_gathers(
    at_hbm, bw_hbm, idx_hbm, zt_hbm, bg_hbm,
    idx_v, zbuf, bbuf, sem_a, sem_b, sem_c, sem_d,
):
    wid = lax.axis_index("s") * NC + lax.axis_index("c")
    base = wid * TOK_PER_W
    pltpu.sync_copy(idx_hbm.at[pl.ds(base, TOK_PER_W)], idx_v)
    _pipelined_gather(at_hbm, idx_v, zt_hbm, base, zbuf, (sem_a, sem_b), (sem_c, sem_d))
    _pipelined_gather(bw_hbm, idx_v, bg_hbm, base, bbuf, (sem_a, sem_b), (sem_c, sem_d))


TBLK = 2048


def _tc_body(zt_ref, bg_ref, b_ref, out_ref):
    acc = lax.dot_general(
        zt_ref[...], b_ref[...],
        (((1,), (1,)), ((), ())),
        preferred_element_type=jnp.float32,
    )
    out_ref[...] = bg_ref[...] + acc * SCALING


def _tc_body_alias(zt_ref, bg_ref, b_ref, prev_ref, out_ref):
    _tc_body(zt_ref, bg_ref, b_ref, out_ref)


def _tc_first(zt, base_g, lora_B):
    return pl.pallas_call(
        _tc_body,
        grid=(HALF // TBLK,),
        in_specs=[
            pl.BlockSpec((TBLK, RANK), lambda i: (i, 0)),
            pl.BlockSpec((TBLK, DIM), lambda i: (i, 0)),
            pl.BlockSpec((DIM, RANK), lambda i: (0, 0)),
        ],
        out_specs=pl.BlockSpec((TBLK, DIM), lambda i: (i, 0)),
        out_shape=jax.ShapeDtypeStruct((N_TOK, DIM), jnp.float32),
    )(zt, base_g, lora_B)


def _tc_second(zt, base_g, lora_B, prev):
    off = HALF // TBLK
    return pl.pallas_call(
        _tc_body_alias,
        grid=(HALF // TBLK,),
        in_specs=[
            pl.BlockSpec((TBLK, RANK), lambda i: (i, 0)),
            pl.BlockSpec((TBLK, DIM), lambda i: (i, 0)),
            pl.BlockSpec((DIM, RANK), lambda i: (0, 0)),
            pl.BlockSpec(memory_space=pl.ANY),
        ],
        out_specs=pl.BlockSpec((TBLK, DIM), lambda i: (i + off, 0)),
        out_shape=jax.ShapeDtypeStruct((N_TOK, DIM), jnp.float32),
        input_output_aliases={3: 0},
    )(zt, base_g, lora_B, prev)


def kernel(input_ids, base_weight, lora_A, lora_B):
    n_i, n_j = input_ids.shape
    ids = input_ids.T.reshape(-1).astype(jnp.int32)
    at = lora_A.T
    zt1, bg1 = _sc_gathers(at, base_weight, ids[:HALF])
    zt2, bg2 = _sc_gathers(at, base_weight, ids[HALF:])
    out1 = _tc_first(zt1, bg1, lora_B)
    out = _tc_second(zt2, bg2, lora_B, out1)
    return out.reshape(n_j, n_i, DIM).swapaxes(0, 1)
